# Initial kernel scaffold; baseline (speedup 1.0000x reference)
import jax
import jax.numpy as jnp
from jax.experimental import pallas as pl


def kernel(x, edge_index, W_l, b_l, W_r):
    raise NotImplementedError("write your pallas kernel here")


# speedup vs baseline: 28.4895x; 28.4895x over previous
import dataclasses
import functools

import jax
import jax.numpy as jnp
from jax import lax
from jax.experimental import pallas as pl
from jax.experimental.pallas import tpu as pltpu
from jax.experimental.pallas import tpu_sc as plsc

N_NODES = 10000
N_EDGES = 320000
D_FEAT = 128
OUT_CH = 4

NC, NS = 2, 16
NW = NC * NS
EPT = N_EDGES // NW
LANES = 16
NV = 10000
CHE = 2000
NCOL = OUT_CH + 1


def _project_body(x_ref, wlt_ref, wrt_ref, bl_ref, y_ref, z_ref):
  x = x_ref[...]
  dn = (((1,), (1,)), ((), ()))
  y_ref[...] = lax.dot_general(
      wlt_ref[...], x, dn, preferred_element_type=jnp.float32)
  zt = lax.dot_general(wrt_ref[...], x, dn, preferred_element_type=jnp.float32)
  z_ref[...] = zt + bl_ref[...]


def _finish_body(acc_ref, z_ref, o_ref):
  s = jnp.sum(acc_ref[...], axis=0)
  deg = jnp.clip(s[OUT_CH:NCOL, :], 1.0, None)
  vals = s[0:OUT_CH, :] / deg + z_ref[...]
  o_ref[...] = jnp.max(vals, axis=1, keepdims=True)


_SC_MESH = plsc.VectorSubcoreMesh(
    core_axis_name="c", subcore_axis_name="s", num_cores=NC, num_subcores=NS
)

_SC_PARAMS = pltpu.CompilerParams()
if "needs_layout_passes" in pltpu.CompilerParams.__dataclass_fields__:
  _SC_PARAMS = dataclasses.replace(_SC_PARAMS, needs_layout_passes=False)


@functools.partial(
    pl.kernel,
    out_type=jax.ShapeDtypeStruct((NW, NCOL, NV), jnp.float32),
    mesh=_SC_MESH,
    scratch_types=[
        pltpu.VMEM((OUT_CH, NV), jnp.float32),
        pltpu.VMEM((NCOL, NV), jnp.float32),
        pltpu.VMEM((CHE,), jnp.int32),
        pltpu.VMEM((CHE,), jnp.int32),
    ],
    compiler_params=_SC_PARAMS,
)
def _edge_kernel(y_hbm, src_hbm, dst_hbm, out_hbm, y_v, acc_v, src_v, dst_v):
  cid = lax.axis_index("c")
  sid = lax.axis_index("s")
  wid = cid * NS + sid

  pltpu.sync_copy(y_hbm, y_v)

  @pl.loop(0, NV, step=LANES)
  def _(i):
    z = jnp.zeros((LANES,), jnp.float32)
    for c in range(NCOL):
      acc_v[c, pl.ds(i, LANES)] = z

  ones = jnp.ones((LANES,), jnp.float32)
  rows = [jnp.full((LANES,), c, jnp.int32) for c in range(NCOL)]

  @pl.loop(0, EPT, step=CHE)
  def _(k):
    base = wid * EPT + k
    pltpu.sync_copy(src_hbm.at[pl.ds(base, CHE)], src_v)
    pltpu.sync_copy(dst_hbm.at[pl.ds(base, CHE)], dst_v)

    @pl.loop(0, CHE, step=LANES)
    def _(j):
      s = src_v[pl.ds(j, LANES)]
      d = dst_v[pl.ds(j, LANES)]
      for c in range(OUT_CH):
        g = plsc.load_gather(y_v, [rows[c], s])
        plsc.addupdate_scatter(acc_v, [rows[c], d], g)
      plsc.addupdate_scatter(acc_v, [rows[OUT_CH], d], ones)

  pltpu.sync_copy(acc_v, out_hbm.at[wid])


@jax.jit
def kernel(x, edge_index, W_l, b_l, W_r):
  wlt = W_l.T
  wrt = W_r.T
  bl = b_l.reshape(OUT_CH, 1)
  src = edge_index[0].astype(jnp.int32)
  dst = edge_index[1].astype(jnp.int32)

  y, z = pl.pallas_call(
      _project_body,
      out_shape=(
          jax.ShapeDtypeStruct((OUT_CH, NV), jnp.float32),
          jax.ShapeDtypeStruct((OUT_CH, NV), jnp.float32),
      ),
  )(x, wlt, wrt, bl)

  acc = _edge_kernel(y, src, dst)

  out = pl.pallas_call(
      _finish_body,
      out_shape=jax.ShapeDtypeStruct((OUT_CH, 1), jnp.float32),
  )(acc, z)
  return out.reshape(OUT_CH)


# speedup vs baseline: 34.1613x; 1.1991x over previous
import dataclasses
import functools

import jax
import jax.numpy as jnp
from jax import lax
from jax.experimental import pallas as pl
from jax.experimental.pallas import tpu as pltpu
from jax.experimental.pallas import tpu_sc as plsc

N_NODES = 10000
N_EDGES = 320000
D_FEAT = 128
OUT_CH = 4

NC, NS = 2, 16
NW = NC * NS
EPT = N_EDGES // NW
LANES = 16
NV = 10000
CHE = 2000
NCOL = OUT_CH + 1


def _project_body(x_ref, wlt_ref, wrt_ref, bl_ref, y_ref, z_ref):
  x = x_ref[...]
  dn = (((1,), (1,)), ((), ()))
  y_ref[...] = lax.dot_general(
      wlt_ref[...], x, dn, preferred_element_type=jnp.float32)
  zt = lax.dot_general(wrt_ref[...], x, dn, preferred_element_type=jnp.float32)
  z_ref[...] = zt + bl_ref[...]


def _finish_body(acc_ref, z_ref, o_ref):
  s = jnp.sum(acc_ref[...], axis=0)
  deg = jnp.clip(s[OUT_CH:NCOL, :], 1.0, None)
  vals = s[0:OUT_CH, :] / deg + z_ref[...]
  o_ref[...] = jnp.max(vals, axis=1, keepdims=True)


_SC_MESH = plsc.VectorSubcoreMesh(
    core_axis_name="c", subcore_axis_name="s", num_cores=NC, num_subcores=NS
)

_SC_PARAMS = pltpu.CompilerParams()
if "needs_layout_passes" in pltpu.CompilerParams.__dataclass_fields__:
  _SC_PARAMS = dataclasses.replace(_SC_PARAMS, needs_layout_passes=False)


@functools.partial(
    pl.kernel,
    out_type=jax.ShapeDtypeStruct((NW, NCOL, NV), jnp.float32),
    mesh=_SC_MESH,
    scratch_types=[
        pltpu.VMEM((OUT_CH, NV), jnp.float32),
        pltpu.VMEM((NCOL, NV), jnp.float32),
        pltpu.VMEM((CHE,), jnp.int32),
        pltpu.VMEM((CHE,), jnp.int32),
        pltpu.SemaphoreType.DMA,
    ],
    compiler_params=_SC_PARAMS,
)
def _edge_kernel(y_hbm, src_hbm, dst_hbm, out_hbm, y_v, acc_v, src_v, dst_v,
                 sem):
  cid = lax.axis_index("c")
  sid = lax.axis_index("s")
  wid = cid * NS + sid

  cp_y = pltpu.async_copy(y_hbm, y_v, sem)

  @plsc.parallel_loop(0, NV, step=LANES, unroll=8)
  def _(i):
    z = jnp.zeros((LANES,), jnp.float32)
    for c in range(NCOL):
      acc_v[c, pl.ds(i, LANES)] = z

  cp_y.wait()

  ones = jnp.ones((LANES,), jnp.float32)
  rows = [jnp.full((LANES,), c, jnp.int32) for c in range(NCOL)]

  @pl.loop(0, EPT, step=CHE)
  def _(k):
    base = wid * EPT + k
    pltpu.sync_copy(src_hbm.at[pl.ds(base, CHE)], src_v)
    pltpu.sync_copy(dst_hbm.at[pl.ds(base, CHE)], dst_v)

    @plsc.parallel_loop(0, CHE, step=LANES, unroll=4)
    def _(j):
      s = src_v[pl.ds(j, LANES)]
      d = dst_v[pl.ds(j, LANES)]
      for c in range(OUT_CH):
        g = plsc.load_gather(y_v, [rows[c], s])
        plsc.addupdate_scatter(acc_v, [rows[c], d], g)
      plsc.addupdate_scatter(acc_v, [rows[OUT_CH], d], ones)

  pltpu.sync_copy(acc_v, out_hbm.at[wid])


@jax.jit
def kernel(x, edge_index, W_l, b_l, W_r):
  wlt = W_l.T
  wrt = W_r.T
  bl = b_l.reshape(OUT_CH, 1)
  src = edge_index[0].astype(jnp.int32)
  dst = edge_index[1].astype(jnp.int32)

  y, z = pl.pallas_call(
      _project_body,
      out_shape=(
          jax.ShapeDtypeStruct((OUT_CH, NV), jnp.float32),
          jax.ShapeDtypeStruct((OUT_CH, NV), jnp.float32),
      ),
  )(x, wlt, wrt, bl)

  acc = _edge_kernel(y, src, dst)

  out = pl.pallas_call(
      _finish_body,
      out_shape=jax.ShapeDtypeStruct((OUT_CH, 1), jnp.float32),
  )(acc, z)
  return out.reshape(OUT_CH)
